# baseline (speedup 1.0000x reference)
import jax
import jax.numpy as jnp
from jax import lax
from jax.experimental import pallas as pl
from jax.experimental.pallas import tpu as pltpu
from jax.experimental.pallas import tpu_sc as plsc

_B, _C, _H, _W = 64, 768, 24, 24
_D = _H * _W
_R = _B * _C
_NC, _NS = 2, 16
_NW = _NC * _NS
_ROWS_PER_W = _R // _NW
_K = 128
_CHUNKS = _ROWS_PER_W // _K


def _sc_body(x_hbm, idx_hbm, out_hbm, idx_v, rows_v, sem):
    wid = lax.axis_index("s") * _NC + lax.axis_index("c")
    base = wid * _ROWS_PER_W
    pltpu.sync_copy(idx_hbm.at[pl.ds(base, _ROWS_PER_W)], idx_v)

    def chunk(i, carry):
        off = i * _K
        pltpu.async_copy(
            x_hbm.at[idx_v.at[pl.ds(off, _K)]], rows_v, sem
        ).wait()
        pltpu.sync_copy(rows_v, out_hbm.at[pl.ds(base + off, _K)])
        return carry

    lax.fori_loop(0, _CHUNKS, chunk, 0)


def kernel(input):
    x = input.reshape(_R, _D)
    r = jnp.arange(_R, dtype=jnp.int32)
    idx = (r // _C) * _C + (_C - 1) - (r % _C)
    run = pl.kernel(
        _sc_body,
        out_type=jax.ShapeDtypeStruct((_R, _D), jnp.float32),
        mesh=plsc.VectorSubcoreMesh(core_axis_name="c", subcore_axis_name="s"),
        scratch_types=[
            pltpu.VMEM((_ROWS_PER_W,), jnp.int32),
            pltpu.VMEM((_K, _D), jnp.float32),
            pltpu.SemaphoreType.DMA,
        ],
        compiler_params=pltpu.CompilerParams(use_tc_tiling_on_sc=False),
    )
    out = run(x, idx)
    return out.reshape(_B, _C, _H, _W)
